# Initial kernel scaffold; baseline (speedup 1.0000x reference)
import jax
import jax.numpy as jnp
from jax.experimental import pallas as pl


def kernel(x, batch):
    raise NotImplementedError("write your pallas kernel here")


# speedup vs baseline: 5.0561x; 5.0561x over previous
import functools

import jax
import jax.numpy as jnp
from jax import lax
from jax.experimental import pallas as pl
from jax.experimental.pallas import tpu as pltpu
from jax.experimental.pallas import tpu_sc as plsc

N_ROWS = 50000
D = 256
G = 128
NC = 2
NS = 16
DH = D // NC
BLK = 128
NFULL = N_ROWS // BLK
REM = N_ROWS - NFULL * BLK
REM_BASE = NFULL * BLK


def _sc_body(x_hbm, b_hbm, out_hbm, xbuf, idxbuf, x2, idx2, acc):
    c = lax.axis_index("c")
    s = lax.axis_index("s")
    col0 = c * DH

    @pl.when(s == 0)
    def _zero():
        zv = jnp.zeros((16,), jnp.float32)

        def zrow(i, _):
            def zcol(j, _):
                xbuf[i, pl.ds(j * 16, 16)] = zv
                return 0

            return lax.fori_loop(0, DH // 16, zcol, 0)

        lax.fori_loop(0, G, zrow, 0)
        pltpu.sync_copy(xbuf, acc)

    plsc.subcore_barrier()

    nblk = (NFULL - s + NS - 1) // NS

    def blk(i, _):
        base = (s + i * NS) * BLK
        pltpu.sync_copy(b_hbm.at[pl.ds(base, BLK)], idxbuf)
        pltpu.sync_copy(x_hbm.at[pl.ds(base, BLK), pl.ds(col0, DH)], xbuf)
        pltpu.sync_copy(xbuf, acc.at[idxbuf], add=True)
        return 0

    lax.fori_loop(0, nblk, blk, 0)

    @pl.when(s == NS - 1)
    def _rem():
        pltpu.sync_copy(b_hbm.at[pl.ds(REM_BASE, REM)], idx2)
        pltpu.sync_copy(x_hbm.at[pl.ds(REM_BASE, REM), pl.ds(col0, DH)], x2)
        pltpu.sync_copy(x2, acc.at[idx2], add=True)

    plsc.subcore_barrier()

    rows = G // NS
    r0 = s * rows
    pltpu.sync_copy(
        acc.at[pl.ds(r0, rows)], out_hbm.at[pl.ds(r0, rows), pl.ds(col0, DH)]
    )


@jax.jit
def _sumpool(x, batch):
    mesh = plsc.VectorSubcoreMesh(core_axis_name="c", subcore_axis_name="s")
    return pl.kernel(
        _sc_body,
        out_type=jax.ShapeDtypeStruct((G, D), jnp.float32),
        mesh=mesh,
        scratch_types=[
            pltpu.VMEM((BLK, DH), jnp.float32),
            pltpu.VMEM((BLK,), jnp.int32),
            pltpu.VMEM((REM, DH), jnp.float32),
            pltpu.VMEM((REM,), jnp.int32),
            pltpu.VMEM_SHARED((G, DH), jnp.float32),
        ],
    )(x, batch)


def kernel(x, batch):
    return _sumpool(x, batch.astype(jnp.int32))


# speedup vs baseline: 8.1193x; 1.6058x over previous
import functools

import jax
import jax.numpy as jnp
from jax import lax
from jax.experimental import pallas as pl
from jax.experimental.pallas import tpu as pltpu
from jax.experimental.pallas import tpu_sc as plsc

N_ROWS = 50000
D = 256
G = 128
NC = 2
NS = 16
DH = D // NC
BLK = 128
NFULL = N_ROWS // BLK
REM = N_ROWS - NFULL * BLK
REM_BASE = NFULL * BLK


def _sc_body(x_hbm, b_hbm, out_hbm, xbuf, idxbuf, x2, idx2, acc, sg, ss):
    c = lax.axis_index("c")
    s = lax.axis_index("s")
    col0 = c * DH

    @pl.when(s == 0)
    def _zero():
        zv = jnp.zeros((16,), jnp.float32)

        def zrow(i, _):
            def zcol(j, _):
                xbuf[0, i, pl.ds(j * 16, 16)] = zv
                return 0

            return lax.fori_loop(0, DH // 16, zcol, 0)

        lax.fori_loop(0, G, zrow, 0)
        pltpu.sync_copy(xbuf.at[0], acc)

    plsc.subcore_barrier()

    nblk = (NFULL - s + NS - 1) // NS

    def base_of(i):
        return (s + i * NS) * BLK

    def fire_gather(i, slot):
        base = base_of(i)
        pltpu.async_copy(b_hbm.at[pl.ds(base, BLK)], idxbuf.at[slot], sg)
        pltpu.async_copy(
            x_hbm.at[pl.ds(base, BLK), pl.ds(col0, DH)], xbuf.at[slot], sg
        )

    def wait_gather(i, slot):
        base = base_of(i)
        pltpu.make_async_copy(
            b_hbm.at[pl.ds(base, BLK)], idxbuf.at[slot], sg
        ).wait()
        pltpu.make_async_copy(
            x_hbm.at[pl.ds(base, BLK), pl.ds(col0, DH)], xbuf.at[slot], sg
        ).wait()

    def wait_scatter(slot):
        pltpu.make_async_copy(xbuf.at[slot], acc.at[idxbuf.at[slot]], ss).wait()

    fire_gather(0, 0)

    def blk(i, _):
        slot = i % 2
        other = 1 - slot

        @pl.when(i >= 1)
        def _():
            wait_scatter(other)

        @pl.when(i + 1 < nblk)
        def _():
            fire_gather(i + 1, other)

        wait_gather(i, slot)
        pltpu.async_copy(xbuf.at[slot], acc.at[idxbuf.at[slot]], ss, add=True)
        return 0

    lax.fori_loop(0, nblk, blk, 0)
    wait_scatter((nblk - 1) % 2)

    @pl.when(s == NS - 1)
    def _rem():
        pltpu.sync_copy(b_hbm.at[pl.ds(REM_BASE, REM)], idx2)
        pltpu.sync_copy(x_hbm.at[pl.ds(REM_BASE, REM), pl.ds(col0, DH)], x2)
        pltpu.sync_copy(x2, acc.at[idx2], add=True)

    plsc.subcore_barrier()

    rows = G // NS
    r0 = s * rows
    pltpu.sync_copy(
        acc.at[pl.ds(r0, rows)], out_hbm.at[pl.ds(r0, rows), pl.ds(col0, DH)]
    )


@jax.jit
def _sumpool(x, batch):
    mesh = plsc.VectorSubcoreMesh(core_axis_name="c", subcore_axis_name="s")
    return pl.kernel(
        _sc_body,
        out_type=jax.ShapeDtypeStruct((G, D), jnp.float32),
        mesh=mesh,
        scratch_types=[
            pltpu.VMEM((2, BLK, DH), jnp.float32),
            pltpu.VMEM((2, BLK), jnp.int32),
            pltpu.VMEM((REM, DH), jnp.float32),
            pltpu.VMEM((REM,), jnp.int32),
            pltpu.VMEM_SHARED((G, DH), jnp.float32),
            pltpu.SemaphoreType.DMA,
            pltpu.SemaphoreType.DMA,
        ],
    )(x, batch)


def kernel(x, batch):
    return _sumpool(x, batch.astype(jnp.int32))


# speedup vs baseline: 8.5272x; 1.0502x over previous
import functools

import jax
import jax.numpy as jnp
from jax import lax
from jax.experimental import pallas as pl
from jax.experimental.pallas import tpu as pltpu
from jax.experimental.pallas import tpu_sc as plsc

N_ROWS = 50000
D = 256
G = 128
NC = 2
NS = 16
DH = D // NC
BLK = 128
NFULL = N_ROWS // BLK
REM = N_ROWS - NFULL * BLK
REM_BASE = NFULL * BLK


DEPTH = 4


def _sc_body(x_hbm, b_hbm, out_hbm, xbuf, idxbuf, x2, idx2, acc, zbuf, sg, ss, s2):
    c = lax.axis_index("c")
    s = lax.axis_index("s")
    col0 = c * DH

    nblk = (NFULL - s + NS - 1) // NS

    def base_of(i):
        return (s + i * NS) * BLK

    def fire_gather(i, slot):
        base = base_of(i)
        pltpu.async_copy(b_hbm.at[pl.ds(base, BLK)], idxbuf.at[slot], sg)
        pltpu.async_copy(
            x_hbm.at[pl.ds(base, BLK), pl.ds(col0, DH)], xbuf.at[slot], sg
        )

    def wait_gather(i, slot):
        base = base_of(i)
        pltpu.make_async_copy(
            b_hbm.at[pl.ds(base, BLK)], idxbuf.at[slot], sg
        ).wait()
        pltpu.make_async_copy(
            x_hbm.at[pl.ds(base, BLK), pl.ds(col0, DH)], xbuf.at[slot], sg
        ).wait()

    def wait_scatter(slot):
        pltpu.make_async_copy(xbuf.at[slot], acc.at[idxbuf.at[slot]], ss).wait()

    for j in range(DEPTH - 1):

        @pl.when(j < nblk)
        def _(j=j):
            fire_gather(j, j)

    @pl.when(s == NS - 1)
    def _rem_fetch():
        pltpu.async_copy(b_hbm.at[pl.ds(REM_BASE, REM)], idx2, s2)
        pltpu.async_copy(x_hbm.at[pl.ds(REM_BASE, REM), pl.ds(col0, DH)], x2, s2)

    zv = jnp.zeros((16,), jnp.float32)
    rows = G // NS

    def zrow(i, _):
        def zcol(j, _):
            zbuf[i, pl.ds(j * 16, 16)] = zv
            return 0

        return lax.fori_loop(0, DH // 16, zcol, 0)

    lax.fori_loop(0, rows, zrow, 0)
    r0 = s * rows
    pltpu.sync_copy(zbuf, acc.at[pl.ds(r0, rows)])

    plsc.subcore_barrier()

    @pl.when(s == NS - 1)
    def _rem_scatter():
        pltpu.make_async_copy(b_hbm.at[pl.ds(REM_BASE, REM)], idx2, s2).wait()
        pltpu.make_async_copy(
            x_hbm.at[pl.ds(REM_BASE, REM), pl.ds(col0, DH)], x2, s2
        ).wait()
        pltpu.async_copy(x2, acc.at[idx2], s2, add=True)

    def blk(i, _):
        slot = i % DEPTH

        @pl.when(i >= 1)
        def _():
            wait_scatter((i - 1) % DEPTH)

        @pl.when(i + DEPTH - 1 < nblk)
        def _():
            fire_gather(i + DEPTH - 1, (i + DEPTH - 1) % DEPTH)

        wait_gather(i, slot)
        pltpu.async_copy(xbuf.at[slot], acc.at[idxbuf.at[slot]], ss, add=True)
        return 0

    lax.fori_loop(0, nblk, blk, 0)
    wait_scatter((nblk - 1) % DEPTH)

    @pl.when(s == NS - 1)
    def _rem_wait():
        pltpu.make_async_copy(x2, acc.at[idx2], s2).wait()

    plsc.subcore_barrier()

    rows = G // NS
    r0 = s * rows
    pltpu.sync_copy(
        acc.at[pl.ds(r0, rows)], out_hbm.at[pl.ds(r0, rows), pl.ds(col0, DH)]
    )


@jax.jit
def _sumpool(x, batch):
    mesh = plsc.VectorSubcoreMesh(core_axis_name="c", subcore_axis_name="s")
    return pl.kernel(
        _sc_body,
        out_type=jax.ShapeDtypeStruct((G, D), jnp.float32),
        mesh=mesh,
        scratch_types=[
            pltpu.VMEM((DEPTH, BLK, DH), jnp.float32),
            pltpu.VMEM((DEPTH, BLK), jnp.int32),
            pltpu.VMEM((REM, DH), jnp.float32),
            pltpu.VMEM((REM,), jnp.int32),
            pltpu.VMEM_SHARED((G, DH), jnp.float32),
            pltpu.VMEM((G // NS, DH), jnp.float32),
            pltpu.SemaphoreType.DMA,
            pltpu.SemaphoreType.DMA,
            pltpu.SemaphoreType.DMA,
        ],
    )(x, batch)


def kernel(x, batch):
    return _sumpool(x, batch.astype(jnp.int32))


# speedup vs baseline: 9.8024x; 1.1495x over previous
import functools

import jax
import jax.numpy as jnp
from jax import lax
from jax.experimental import pallas as pl
from jax.experimental.pallas import tpu as pltpu
from jax.experimental.pallas import tpu_sc as plsc

N_ROWS = 50000
D = 256
G = 128
NC = 2
NS = 16
DH = D // NC
BLK = 128
NFULL = N_ROWS // BLK
REM = N_ROWS - NFULL * BLK
REM_BASE = NFULL * BLK

TC_BLK = 2048
TC_GRID = 9
N_TC = TC_GRID * TC_BLK
SC_B0 = N_TC // BLK
SC_NFULL = NFULL - SC_B0


DEPTH = 4


def _sc_body(x_hbm, b_hbm, out_hbm, xbuf, idxbuf, x2, idx2, acc, zbuf, sg, ss, s2):
    c = lax.axis_index("c")
    s = lax.axis_index("s")
    col0 = c * DH

    nblk = (SC_NFULL - s + NS - 1) // NS

    def base_of(i):
        return (SC_B0 + s + i * NS) * BLK

    def fire_gather(i, slot):
        base = base_of(i)
        pltpu.async_copy(b_hbm.at[pl.ds(base, BLK)], idxbuf.at[slot], sg)
        pltpu.async_copy(
            x_hbm.at[pl.ds(base, BLK), pl.ds(col0, DH)], xbuf.at[slot], sg
        )

    def wait_gather(i, slot):
        base = base_of(i)
        pltpu.make_async_copy(
            b_hbm.at[pl.ds(base, BLK)], idxbuf.at[slot], sg
        ).wait()
        pltpu.make_async_copy(
            x_hbm.at[pl.ds(base, BLK), pl.ds(col0, DH)], xbuf.at[slot], sg
        ).wait()

    def wait_scatter(slot):
        pltpu.make_async_copy(xbuf.at[slot], acc.at[idxbuf.at[slot]], ss).wait()

    for j in range(DEPTH - 1):

        @pl.when(j < nblk)
        def _(j=j):
            fire_gather(j, j)

    @pl.when(s == NS - 1)
    def _rem_fetch():
        pltpu.async_copy(b_hbm.at[pl.ds(REM_BASE, REM)], idx2, s2)
        pltpu.async_copy(x_hbm.at[pl.ds(REM_BASE, REM), pl.ds(col0, DH)], x2, s2)

    zv = jnp.zeros((16,), jnp.float32)
    rows = G // NS

    def zrow(i, _):
        def zcol(j, _):
            zbuf[i, pl.ds(j * 16, 16)] = zv
            return 0

        return lax.fori_loop(0, DH // 16, zcol, 0)

    lax.fori_loop(0, rows, zrow, 0)
    r0 = s * rows
    pltpu.sync_copy(zbuf, acc.at[pl.ds(r0, rows)])

    plsc.subcore_barrier()

    @pl.when(s == NS - 1)
    def _rem_scatter():
        pltpu.make_async_copy(b_hbm.at[pl.ds(REM_BASE, REM)], idx2, s2).wait()
        pltpu.make_async_copy(
            x_hbm.at[pl.ds(REM_BASE, REM), pl.ds(col0, DH)], x2, s2
        ).wait()
        pltpu.async_copy(x2, acc.at[idx2], s2, add=True)

    def blk(i, _):
        slot = i % DEPTH

        @pl.when(i >= 1)
        def _():
            wait_scatter((i - 1) % DEPTH)

        @pl.when(i + DEPTH - 1 < nblk)
        def _():
            fire_gather(i + DEPTH - 1, (i + DEPTH - 1) % DEPTH)

        wait_gather(i, slot)
        pltpu.async_copy(xbuf.at[slot], acc.at[idxbuf.at[slot]], ss, add=True)
        return 0

    lax.fori_loop(0, nblk, blk, 0)
    wait_scatter((nblk - 1) % DEPTH)

    @pl.when(s == NS - 1)
    def _rem_wait():
        pltpu.make_async_copy(x2, acc.at[idx2], s2).wait()

    plsc.subcore_barrier()

    rows = G // NS
    r0 = s * rows
    pltpu.sync_copy(
        acc.at[pl.ds(r0, rows)], out_hbm.at[pl.ds(r0, rows), pl.ds(col0, DH)]
    )


def _tc_body(b_ref, x_ref, out_ref):
    j = pl.program_id(0)
    bb = b_ref[0, 0, :]
    onehot = (
        lax.broadcasted_iota(jnp.int32, (G, TC_BLK), 0) == bb[None, :]
    ).astype(jnp.float32)
    p = jnp.dot(onehot, x_ref[...], preferred_element_type=jnp.float32)

    @pl.when(j == 0)
    def _():
        out_ref[...] = p

    @pl.when(j > 0)
    def _():
        out_ref[...] += p


def _add_body(a_ref, b_ref, o_ref):
    o_ref[...] = a_ref[...] + b_ref[...]


@jax.jit
def _sumpool(x, batch):
    mesh = plsc.VectorSubcoreMesh(core_axis_name="c", subcore_axis_name="s")
    part_sc = pl.kernel(
        _sc_body,
        out_type=jax.ShapeDtypeStruct((G, D), jnp.float32),
        mesh=mesh,
        scratch_types=[
            pltpu.VMEM((DEPTH, BLK, DH), jnp.float32),
            pltpu.VMEM((DEPTH, BLK), jnp.int32),
            pltpu.VMEM((REM, DH), jnp.float32),
            pltpu.VMEM((REM,), jnp.int32),
            pltpu.VMEM_SHARED((G, DH), jnp.float32),
            pltpu.VMEM((G // NS, DH), jnp.float32),
            pltpu.SemaphoreType.DMA,
            pltpu.SemaphoreType.DMA,
            pltpu.SemaphoreType.DMA,
        ],
    )(x, batch)

    b3 = batch[:N_TC].reshape(TC_GRID, 1, TC_BLK)
    part_tc = pl.pallas_call(
        _tc_body,
        grid=(TC_GRID,),
        in_specs=[
            pl.BlockSpec((1, 1, TC_BLK), lambda j: (j, 0, 0)),
            pl.BlockSpec((TC_BLK, D), lambda j: (j, 0)),
        ],
        out_specs=pl.BlockSpec((G, D), lambda j: (0, 0)),
        out_shape=jax.ShapeDtypeStruct((G, D), jnp.float32),
    )(b3, x)

    return pl.pallas_call(
        _add_body,
        out_shape=jax.ShapeDtypeStruct((G, D), jnp.float32),
    )(part_sc, part_tc)


def kernel(x, batch):
    return _sumpool(x, batch.astype(jnp.int32))


# speedup vs baseline: 10.2597x; 1.0467x over previous
import functools

import jax
import jax.numpy as jnp
from jax import lax
from jax.experimental import pallas as pl
from jax.experimental.pallas import tpu as pltpu
from jax.experimental.pallas import tpu_sc as plsc

N_ROWS = 50000
D = 256
G = 128
NC = 2
NS = 16
DH = D // NC
BLK = 128
NFULL = N_ROWS // BLK
REM = N_ROWS - NFULL * BLK
REM_BASE = NFULL * BLK

TC_BLK = 4096
TC_GRID = 6
N_TC = TC_GRID * TC_BLK
SC_B0 = N_TC // BLK
SC_NFULL = NFULL - SC_B0


DEPTH = 4


def _sc_body(x_hbm, b_hbm, out_hbm, xbuf, idxbuf, x2, idx2, acc, zbuf, sg, ss, s2):
    c = lax.axis_index("c")
    s = lax.axis_index("s")
    col0 = c * DH

    nblk = (SC_NFULL - s + NS - 1) // NS

    def base_of(i):
        return (SC_B0 + s + i * NS) * BLK

    def fire_gather(i, slot):
        base = base_of(i)
        pltpu.async_copy(b_hbm.at[pl.ds(base, BLK)], idxbuf.at[slot], sg)
        pltpu.async_copy(
            x_hbm.at[pl.ds(base, BLK), pl.ds(col0, DH)], xbuf.at[slot], sg
        )

    def wait_gather(i, slot):
        base = base_of(i)
        pltpu.make_async_copy(
            b_hbm.at[pl.ds(base, BLK)], idxbuf.at[slot], sg
        ).wait()
        pltpu.make_async_copy(
            x_hbm.at[pl.ds(base, BLK), pl.ds(col0, DH)], xbuf.at[slot], sg
        ).wait()

    def wait_scatter(slot):
        pltpu.make_async_copy(xbuf.at[slot], acc.at[idxbuf.at[slot]], ss).wait()

    for j in range(DEPTH - 1):

        @pl.when(j < nblk)
        def _(j=j):
            fire_gather(j, j)

    @pl.when(s == NS - 1)
    def _rem_fetch():
        pltpu.async_copy(b_hbm.at[pl.ds(REM_BASE, REM)], idx2, s2)
        pltpu.async_copy(x_hbm.at[pl.ds(REM_BASE, REM), pl.ds(col0, DH)], x2, s2)

    zv = jnp.zeros((16,), jnp.float32)
    rows = G // NS

    def zrow(i, _):
        def zcol(j, _):
            zbuf[i, pl.ds(j * 16, 16)] = zv
            return 0

        return lax.fori_loop(0, DH // 16, zcol, 0)

    lax.fori_loop(0, rows, zrow, 0)
    r0 = s * rows
    pltpu.sync_copy(zbuf, acc.at[pl.ds(r0, rows)])

    plsc.subcore_barrier()

    @pl.when(s == NS - 1)
    def _rem_scatter():
        pltpu.make_async_copy(b_hbm.at[pl.ds(REM_BASE, REM)], idx2, s2).wait()
        pltpu.make_async_copy(
            x_hbm.at[pl.ds(REM_BASE, REM), pl.ds(col0, DH)], x2, s2
        ).wait()
        pltpu.async_copy(x2, acc.at[idx2], s2, add=True)

    def blk(i, _):
        slot = i % DEPTH

        @pl.when(i >= 1)
        def _():
            wait_scatter((i - 1) % DEPTH)

        @pl.when(i + DEPTH - 1 < nblk)
        def _():
            fire_gather(i + DEPTH - 1, (i + DEPTH - 1) % DEPTH)

        wait_gather(i, slot)
        pltpu.async_copy(xbuf.at[slot], acc.at[idxbuf.at[slot]], ss, add=True)
        return 0

    lax.fori_loop(0, nblk, blk, 0)
    wait_scatter((nblk - 1) % DEPTH)

    @pl.when(s == NS - 1)
    def _rem_wait():
        pltpu.make_async_copy(x2, acc.at[idx2], s2).wait()

    plsc.subcore_barrier()

    rows = G // NS
    r0 = s * rows
    pltpu.sync_copy(
        acc.at[pl.ds(r0, rows)], out_hbm.at[pl.ds(r0, rows), pl.ds(col0, DH)]
    )


def _tc_body(b_ref, x_ref, out_ref):
    j = pl.program_id(0)
    bb = b_ref[...]
    onehot = (
        lax.broadcasted_iota(jnp.int32, (G, TC_BLK), 0) == bb[None, :]
    ).astype(jnp.float32)
    p = jnp.dot(onehot, x_ref[...], preferred_element_type=jnp.float32)

    @pl.when(j == 0)
    def _():
        out_ref[...] = p

    @pl.when(j > 0)
    def _():
        out_ref[...] += p


def _add_body(a_ref, b_ref, o_ref):
    o_ref[...] = a_ref[...] + b_ref[...]


@jax.jit
def _sumpool(x, batch):
    mesh = plsc.VectorSubcoreMesh(core_axis_name="c", subcore_axis_name="s")
    part_sc = pl.kernel(
        _sc_body,
        out_type=jax.ShapeDtypeStruct((G, D), jnp.float32),
        mesh=mesh,
        scratch_types=[
            pltpu.VMEM((DEPTH, BLK, DH), jnp.float32),
            pltpu.VMEM((DEPTH, BLK), jnp.int32),
            pltpu.VMEM((REM, DH), jnp.float32),
            pltpu.VMEM((REM,), jnp.int32),
            pltpu.VMEM_SHARED((G, DH), jnp.float32),
            pltpu.VMEM((G // NS, DH), jnp.float32),
            pltpu.SemaphoreType.DMA,
            pltpu.SemaphoreType.DMA,
            pltpu.SemaphoreType.DMA,
        ],
    )(x, batch)

    part_tc = pl.pallas_call(
        _tc_body,
        grid=(TC_GRID,),
        in_specs=[
            pl.BlockSpec((TC_BLK,), lambda j: (j,)),
            pl.BlockSpec((TC_BLK, D), lambda j: (j, 0)),
        ],
        out_specs=pl.BlockSpec((G, D), lambda j: (0, 0)),
        out_shape=jax.ShapeDtypeStruct((G, D), jnp.float32),
    )(batch, x)

    return pl.pallas_call(
        _add_body,
        out_shape=jax.ShapeDtypeStruct((G, D), jnp.float32),
    )(part_sc, part_tc)


def kernel(x, batch):
    return _sumpool(x, batch.astype(jnp.int32))


# speedup vs baseline: 11.0825x; 1.0802x over previous
import functools

import jax
import jax.numpy as jnp
from jax import lax
from jax.experimental import pallas as pl
from jax.experimental.pallas import tpu as pltpu
from jax.experimental.pallas import tpu_sc as plsc

N_ROWS = 50000
D = 256
G = 128
NC = 2
NS = 16
DH = D // NC
BLK = 128
NFULL = N_ROWS // BLK
REM = N_ROWS - NFULL * BLK
REM_BASE = NFULL * BLK

TC_BLK = 3072
TC_GRID = 10
N_TC = TC_GRID * TC_BLK
SC_B0 = N_TC // BLK
SC_NFULL = NFULL - SC_B0


DEPTH = 4


def _sc_body(x_hbm, b_hbm, out_hbm, xbuf, idxbuf, x2, idx2, acc, zbuf, sg, ss, s2):
    c = lax.axis_index("c")
    s = lax.axis_index("s")
    col0 = c * DH

    nblk = (SC_NFULL - s + NS - 1) // NS

    def base_of(i):
        return (SC_B0 + s + i * NS) * BLK

    def fire_gather(i, slot):
        base = base_of(i)
        pltpu.async_copy(b_hbm.at[pl.ds(base, BLK)], idxbuf.at[slot], sg)
        pltpu.async_copy(
            x_hbm.at[pl.ds(base, BLK), pl.ds(col0, DH)], xbuf.at[slot], sg
        )

    def wait_gather(i, slot):
        base = base_of(i)
        pltpu.make_async_copy(
            b_hbm.at[pl.ds(base, BLK)], idxbuf.at[slot], sg
        ).wait()
        pltpu.make_async_copy(
            x_hbm.at[pl.ds(base, BLK), pl.ds(col0, DH)], xbuf.at[slot], sg
        ).wait()

    def wait_scatter(slot):
        pltpu.make_async_copy(xbuf.at[slot], acc.at[idxbuf.at[slot]], ss).wait()

    for j in range(DEPTH - 1):

        @pl.when(j < nblk)
        def _(j=j):
            fire_gather(j, j)

    @pl.when(s == NS - 1)
    def _rem_fetch():
        pltpu.async_copy(b_hbm.at[pl.ds(REM_BASE, REM)], idx2, s2)
        pltpu.async_copy(x_hbm.at[pl.ds(REM_BASE, REM), pl.ds(col0, DH)], x2, s2)

    zv = jnp.zeros((16,), jnp.float32)
    rows = G // NS

    def zrow(i, _):
        def zcol(j, _):
            zbuf[i, pl.ds(j * 16, 16)] = zv
            return 0

        return lax.fori_loop(0, DH // 16, zcol, 0)

    lax.fori_loop(0, rows, zrow, 0)
    r0 = s * rows
    pltpu.sync_copy(zbuf, acc.at[pl.ds(r0, rows)])

    plsc.subcore_barrier()

    @pl.when(s == NS - 1)
    def _rem_scatter():
        pltpu.make_async_copy(b_hbm.at[pl.ds(REM_BASE, REM)], idx2, s2).wait()
        pltpu.make_async_copy(
            x_hbm.at[pl.ds(REM_BASE, REM), pl.ds(col0, DH)], x2, s2
        ).wait()
        pltpu.async_copy(x2, acc.at[idx2], s2, add=True)

    def blk(i, _):
        slot = i % DEPTH

        @pl.when(i >= 1)
        def _():
            wait_scatter((i - 1) % DEPTH)

        @pl.when(i + DEPTH - 1 < nblk)
        def _():
            fire_gather(i + DEPTH - 1, (i + DEPTH - 1) % DEPTH)

        wait_gather(i, slot)
        pltpu.async_copy(xbuf.at[slot], acc.at[idxbuf.at[slot]], ss, add=True)
        return 0

    lax.fori_loop(0, nblk, blk, 0)
    wait_scatter((nblk - 1) % DEPTH)

    @pl.when(s == NS - 1)
    def _rem_wait():
        pltpu.make_async_copy(x2, acc.at[idx2], s2).wait()

    plsc.subcore_barrier()

    rows = G // NS
    r0 = s * rows
    pltpu.sync_copy(
        acc.at[pl.ds(r0, rows)], out_hbm.at[pl.ds(r0, rows), pl.ds(col0, DH)]
    )


def _tc_body(b_ref, x_ref, out_ref):
    j = pl.program_id(0)
    bb = b_ref[...]
    onehot = (
        lax.broadcasted_iota(jnp.int32, (G, TC_BLK), 0) == bb[None, :]
    ).astype(jnp.float32)
    p = jnp.dot(onehot, x_ref[...], preferred_element_type=jnp.float32)

    @pl.when(j == 0)
    def _():
        out_ref[...] = p

    @pl.when(j > 0)
    def _():
        out_ref[...] += p


def _add_body(a_ref, b_ref, o_ref):
    o_ref[...] = a_ref[...] + b_ref[...]


@jax.jit
def _sumpool(x, batch):
    mesh = plsc.VectorSubcoreMesh(core_axis_name="c", subcore_axis_name="s")
    part_sc = pl.kernel(
        _sc_body,
        out_type=jax.ShapeDtypeStruct((G, D), jnp.float32),
        mesh=mesh,
        scratch_types=[
            pltpu.VMEM((DEPTH, BLK, DH), jnp.float32),
            pltpu.VMEM((DEPTH, BLK), jnp.int32),
            pltpu.VMEM((REM, DH), jnp.float32),
            pltpu.VMEM((REM,), jnp.int32),
            pltpu.VMEM_SHARED((G, DH), jnp.float32),
            pltpu.VMEM((G // NS, DH), jnp.float32),
            pltpu.SemaphoreType.DMA,
            pltpu.SemaphoreType.DMA,
            pltpu.SemaphoreType.DMA,
        ],
    )(x, batch)

    part_tc = pl.pallas_call(
        _tc_body,
        grid=(TC_GRID,),
        in_specs=[
            pl.BlockSpec((TC_BLK,), lambda j: (j,)),
            pl.BlockSpec((TC_BLK, D), lambda j: (j, 0)),
        ],
        out_specs=pl.BlockSpec((G, D), lambda j: (0, 0)),
        out_shape=jax.ShapeDtypeStruct((G, D), jnp.float32),
    )(batch, x)

    return pl.pallas_call(
        _add_body,
        out_shape=jax.ShapeDtypeStruct((G, D), jnp.float32),
    )(part_sc, part_tc)


def kernel(x, batch):
    return _sumpool(x, batch.astype(jnp.int32))
